# baseline (speedup 1.0000x reference)
import functools

import numpy as np
import jax
import jax.numpy as jnp
from jax import lax
from jax.experimental import pallas as pl
from jax.experimental.pallas import tpu as pltpu
from jax.experimental.pallas import tpu_sc as plsc

_N = 4000000
_DIM = 2
_OMEGA_BATCH = 32768
_BORDER_BATCH = 1024
_N_FACETS = 2 * _DIM
_FACET_PTS = 40000 // _N_FACETS

_NC, _NS = 2, 16
_NW = _NC * _NS
_CHUNK = 128


_ROT_A = (13, 15, 26, 6)
_ROT_B = (17, 29, 16, 24)


def _rotl(x, d):
    return (x << np.uint32(d)) | (x >> np.uint32(32 - d))


def _tf2x32(k1, k2, x0, x1):
    k1 = np.uint32(k1)
    k2 = np.uint32(k2)
    ks = (k1, k2, np.uint32(k1 ^ k2 ^ np.uint32(0x1BD11BDA)))
    x0 = (x0 + ks[0]).astype(np.uint32)
    x1 = (x1 + ks[1]).astype(np.uint32)
    rots = (_ROT_A, _ROT_B, _ROT_A, _ROT_B, _ROT_A)
    for i in range(5):
        for r in rots[i]:
            x0 = (x0 + x1).astype(np.uint32)
            x1 = _rotl(x1, r)
            x1 = x1 ^ x0
        x0 = (x0 + ks[(i + 1) % 3]).astype(np.uint32)
        x1 = (x1 + ks[(i + 2) % 3] + np.uint32(i + 1)).astype(np.uint32)
    return x0, x1


def _np_split(key, n):
    lo = np.arange(n, dtype=np.uint32)
    hi = np.zeros(n, dtype=np.uint32)
    b1, b2 = _tf2x32(key[0], key[1], hi, lo)
    return np.stack([b1, b2], axis=1)


def _np_bits32(key, n):
    lo = np.arange(n, dtype=np.uint32)
    hi = np.zeros(n, dtype=np.uint32)
    b1, b2 = _tf2x32(key[0], key[1], hi, lo)
    return b1 ^ b2


def _perm_prefix_np(subkey, n, batch, num_rounds):
    k = subkey
    argsorts = []
    for _ in range(num_rounds):
        pair = _np_split(k, 2)
        k, sk = pair[0], pair[1]
        bits = _np_bits32(sk, n)
        argsorts.append(np.argsort(bits, kind="stable"))
    idx = argsorts[-1][:batch]
    for s in reversed(argsorts[:-1]):
        idx = s[idx]
    return idx


def _const_flat_indices():
    key = np.array([0, 42], dtype=np.uint32)
    key, k_in, k_b = _np_split(key, 3)
    sub = _np_split(k_in, 2)[1]
    idx_o = _perm_prefix_np(sub, _N, _OMEGA_BATCH, 3)
    subb = _np_split(k_b, 2)[1]
    idx_b = _perm_prefix_np(subb, _FACET_PTS, _BORDER_BATCH, 2)
    flat_o = (idx_o.astype(np.int64)[:, None] * _DIM
              + np.arange(_DIM, dtype=np.int64)).reshape(-1).astype(np.int32)
    n_cols_b = _DIM * _N_FACETS
    flat_b = (idx_b.astype(np.int64)[:, None]
              + np.arange(n_cols_b, dtype=np.int64) * _FACET_PTS).reshape(-1).astype(np.int32)
    flat_o = flat_o.reshape(_NW, -1, _CHUNK)
    flat_b = flat_b.reshape(_NW, -1, _CHUNK)
    return flat_o, flat_b


_FLAT_O, _FLAT_B = _const_flat_indices()


def _make_gather_kernel(o_chunks, b_chunks):
    mesh = plsc.VectorSubcoreMesh(core_axis_name="c", subcore_axis_name="s")

    @functools.partial(
        pl.kernel,
        mesh=mesh,
        out_type=[
            jax.ShapeDtypeStruct((_NW, o_chunks, _CHUNK), jnp.float32),
            jax.ShapeDtypeStruct((_NW, b_chunks, _CHUNK), jnp.float32),
        ],
        scratch_types=[
            pltpu.VMEM((o_chunks, _CHUNK), jnp.int32),
            pltpu.VMEM((o_chunks, _CHUNK), jnp.float32),
            pltpu.VMEM((b_chunks, _CHUNK), jnp.int32),
            pltpu.VMEM((b_chunks, _CHUNK), jnp.float32),
            pltpu.SemaphoreType.DMA,
        ],
    )
    def gather_k(omega_hbm, oidx_hbm, border_hbm, bidx_hbm,
                 out_o, out_b, oidx_v, orows_v, bidx_v, brows_v, sem):
        wid = lax.axis_index("s") * _NC + lax.axis_index("c")
        pltpu.sync_copy(oidx_hbm.at[wid], oidx_v)
        pltpu.sync_copy(bidx_hbm.at[wid], bidx_v)
        handles = []
        for j in range(o_chunks):
            handles.append(
                pltpu.async_copy(omega_hbm.at[oidx_v.at[j]], orows_v.at[j], sem))
        for j in range(b_chunks):
            handles.append(
                pltpu.async_copy(border_hbm.at[bidx_v.at[j]], brows_v.at[j], sem))
        for h in handles:
            h.wait()
        pltpu.sync_copy(orows_v, out_o.at[wid])
        pltpu.sync_copy(brows_v, out_b.at[wid])

    return gather_k


def kernel(omega, omega_border):
    flat_o, flat_b = _FLAT_O, _FLAT_B
    o_chunks, b_chunks = flat_o.shape[1], flat_b.shape[1]
    gather_k = _make_gather_kernel(o_chunks, b_chunks)
    omega_u64 = lax.optimization_barrier(
        lax.bitcast_convert_type(omega, jnp.uint64))
    omega_flat = lax.bitcast_convert_type(omega_u64, jnp.float32).reshape(-1)
    border_flat = omega_border.transpose(1, 2, 0).reshape(-1)
    out_o, out_b = gather_k(
        omega_flat,
        jnp.asarray(flat_o),
        border_flat,
        jnp.asarray(flat_b),
    )
    omega_batch = out_o.reshape(_OMEGA_BATCH, _DIM)
    border_batch = out_b.reshape(_BORDER_BATCH, _DIM, _N_FACETS)
    return (omega_batch, border_batch)


# speedup vs baseline: 8.2140x; 8.2140x over previous
import functools

import numpy as np
import jax
import jax.numpy as jnp
from jax import lax
from jax.experimental import pallas as pl
from jax.experimental.pallas import tpu as pltpu
from jax.experimental.pallas import tpu_sc as plsc

_N = 4000000
_DIM = 2
_OMEGA_BATCH = 32768
_BORDER_BATCH = 1024
_N_FACETS = 2 * _DIM
_FACET_PTS = 40000 // _N_FACETS

_NC, _NS = 2, 16
_NW = _NC * _NS
_CHUNK = 128


_ROT_A = (13, 15, 26, 6)
_ROT_B = (17, 29, 16, 24)


def _rotl(x, d):
    return (x << np.uint32(d)) | (x >> np.uint32(32 - d))


def _tf2x32(k1, k2, x0, x1):
    k1 = np.uint32(k1)
    k2 = np.uint32(k2)
    ks = (k1, k2, np.uint32(k1 ^ k2 ^ np.uint32(0x1BD11BDA)))
    x0 = (x0 + ks[0]).astype(np.uint32)
    x1 = (x1 + ks[1]).astype(np.uint32)
    rots = (_ROT_A, _ROT_B, _ROT_A, _ROT_B, _ROT_A)
    for i in range(5):
        for r in rots[i]:
            x0 = (x0 + x1).astype(np.uint32)
            x1 = _rotl(x1, r)
            x1 = x1 ^ x0
        x0 = (x0 + ks[(i + 1) % 3]).astype(np.uint32)
        x1 = (x1 + ks[(i + 2) % 3] + np.uint32(i + 1)).astype(np.uint32)
    return x0, x1


def _np_split(key, n):
    lo = np.arange(n, dtype=np.uint32)
    hi = np.zeros(n, dtype=np.uint32)
    b1, b2 = _tf2x32(key[0], key[1], hi, lo)
    return np.stack([b1, b2], axis=1)


def _np_bits32(key, n):
    lo = np.arange(n, dtype=np.uint32)
    hi = np.zeros(n, dtype=np.uint32)
    b1, b2 = _tf2x32(key[0], key[1], hi, lo)
    return b1 ^ b2


def _perm_prefix_np(subkey, n, batch, num_rounds):
    k = subkey
    argsorts = []
    for _ in range(num_rounds):
        pair = _np_split(k, 2)
        k, sk = pair[0], pair[1]
        bits = _np_bits32(sk, n)
        argsorts.append(np.argsort(bits, kind="stable"))
    idx = argsorts[-1][:batch]
    for s in reversed(argsorts[:-1]):
        idx = s[idx]
    return idx


def _const_flat_indices():
    key = np.array([0, 42], dtype=np.uint32)
    key, k_in, k_b = _np_split(key, 3)
    sub = _np_split(k_in, 2)[1]
    idx_o = _perm_prefix_np(sub, _N, _OMEGA_BATCH, 3)
    subb = _np_split(k_b, 2)[1]
    idx_b = _perm_prefix_np(subb, _FACET_PTS, _BORDER_BATCH, 2)
    flat_o = (idx_o.astype(np.int64)[:, None]
              + np.arange(_DIM, dtype=np.int64) * _N).reshape(-1).astype(np.int32)
    n_cols_b = _DIM * _N_FACETS
    flat_b = (idx_b.astype(np.int64)[:, None]
              + np.arange(n_cols_b, dtype=np.int64) * _FACET_PTS).reshape(-1).astype(np.int32)
    flat_o = flat_o.reshape(_NW, -1, _CHUNK)
    flat_b = flat_b.reshape(_NW, -1, _CHUNK)
    return flat_o, flat_b


_FLAT_O, _FLAT_B = _const_flat_indices()


def _make_gather_kernel(o_chunks, b_chunks):
    mesh = plsc.VectorSubcoreMesh(core_axis_name="c", subcore_axis_name="s")

    @functools.partial(
        pl.kernel,
        mesh=mesh,
        out_type=[
            jax.ShapeDtypeStruct((_NW, o_chunks, _CHUNK), jnp.float32),
            jax.ShapeDtypeStruct((_NW, b_chunks, _CHUNK), jnp.float32),
        ],
        scratch_types=[
            pltpu.VMEM((o_chunks, _CHUNK), jnp.int32),
            pltpu.VMEM((o_chunks, _CHUNK), jnp.float32),
            pltpu.VMEM((b_chunks, _CHUNK), jnp.int32),
            pltpu.VMEM((b_chunks, _CHUNK), jnp.float32),
            pltpu.SemaphoreType.DMA,
        ],
    )
    def gather_k(omega_hbm, oidx_hbm, border_hbm, bidx_hbm,
                 out_o, out_b, oidx_v, orows_v, bidx_v, brows_v, sem):
        wid = lax.axis_index("s") * _NC + lax.axis_index("c")
        pltpu.sync_copy(oidx_hbm.at[wid], oidx_v)
        pltpu.sync_copy(bidx_hbm.at[wid], bidx_v)
        handles = []
        for j in range(o_chunks):
            handles.append(
                pltpu.async_copy(omega_hbm.at[oidx_v.at[j]], orows_v.at[j], sem))
        for j in range(b_chunks):
            handles.append(
                pltpu.async_copy(border_hbm.at[bidx_v.at[j]], brows_v.at[j], sem))
        for h in handles:
            h.wait()
        pltpu.sync_copy(orows_v, out_o.at[wid])
        pltpu.sync_copy(brows_v, out_b.at[wid])

    return gather_k


def kernel(omega, omega_border):
    flat_o, flat_b = _FLAT_O, _FLAT_B
    o_chunks, b_chunks = flat_o.shape[1], flat_b.shape[1]
    gather_k = _make_gather_kernel(o_chunks, b_chunks)
    omega_flat = lax.reshape(omega, (_N * _DIM,), dimensions=(1, 0))
    border_flat = omega_border.transpose(1, 2, 0).reshape(-1)
    out_o, out_b = gather_k(
        omega_flat,
        jnp.asarray(flat_o),
        border_flat,
        jnp.asarray(flat_b),
    )
    omega_batch = out_o.reshape(_OMEGA_BATCH, _DIM)
    border_batch = out_b.reshape(_BORDER_BATCH, _DIM, _N_FACETS)
    return (omega_batch, border_batch)


# speedup vs baseline: 27.7384x; 3.3770x over previous
import functools

import numpy as np
import jax
import jax.numpy as jnp
from jax import lax
from jax.experimental import pallas as pl
from jax.experimental.pallas import tpu as pltpu
from jax.experimental.pallas import tpu_sc as plsc

_N = 4000000
_DIM = 2
_OMEGA_BATCH = 32768
_BORDER_BATCH = 1024
_N_FACETS = 2 * _DIM
_FACET_PTS = 40000 // _N_FACETS

_NC, _NS = 2, 16
_NW = _NC * _NS
_CHUNK = 128


_ROT_A = (13, 15, 26, 6)
_ROT_B = (17, 29, 16, 24)


def _rotl(x, d):
    return (x << np.uint32(d)) | (x >> np.uint32(32 - d))


def _tf2x32(k1, k2, x0, x1):
    k1 = np.uint32(k1)
    k2 = np.uint32(k2)
    ks = (k1, k2, np.uint32(k1 ^ k2 ^ np.uint32(0x1BD11BDA)))
    x0 = (x0 + ks[0]).astype(np.uint32)
    x1 = (x1 + ks[1]).astype(np.uint32)
    rots = (_ROT_A, _ROT_B, _ROT_A, _ROT_B, _ROT_A)
    for i in range(5):
        for r in rots[i]:
            x0 = (x0 + x1).astype(np.uint32)
            x1 = _rotl(x1, r)
            x1 = x1 ^ x0
        x0 = (x0 + ks[(i + 1) % 3]).astype(np.uint32)
        x1 = (x1 + ks[(i + 2) % 3] + np.uint32(i + 1)).astype(np.uint32)
    return x0, x1


def _np_split(key, n):
    lo = np.arange(n, dtype=np.uint32)
    hi = np.zeros(n, dtype=np.uint32)
    b1, b2 = _tf2x32(key[0], key[1], hi, lo)
    return np.stack([b1, b2], axis=1)


def _np_bits32(key, n):
    lo = np.arange(n, dtype=np.uint32)
    hi = np.zeros(n, dtype=np.uint32)
    b1, b2 = _tf2x32(key[0], key[1], hi, lo)
    return b1 ^ b2


def _perm_prefix_np(subkey, n, batch, num_rounds):
    k = subkey
    argsorts = []
    for _ in range(num_rounds):
        pair = _np_split(k, 2)
        k, sk = pair[0], pair[1]
        bits = _np_bits32(sk, n)
        argsorts.append(np.argsort(bits, kind="stable"))
    idx = argsorts[-1][:batch]
    for s in reversed(argsorts[:-1]):
        idx = s[idx]
    return idx


def _const_flat_indices():
    key = np.array([0, 42], dtype=np.uint32)
    key, k_in, k_b = _np_split(key, 3)
    sub = _np_split(k_in, 2)[1]
    idx_o = _perm_prefix_np(sub, _N, _OMEGA_BATCH, 3)
    subb = _np_split(k_b, 2)[1]
    idx_b = _perm_prefix_np(subb, _FACET_PTS, _BORDER_BATCH, 2)
    i64 = idx_o.astype(np.int64)
    flat_o = ((i64 // 128)[:, None] * 256
              + np.arange(_DIM, dtype=np.int64) * 128
              + (i64 % 128)[:, None]).reshape(-1).astype(np.int32)
    n_cols_b = _DIM * _N_FACETS
    flat_b = (idx_b.astype(np.int64)[:, None]
              + np.arange(n_cols_b, dtype=np.int64) * _FACET_PTS).reshape(-1).astype(np.int32)
    flat_o = flat_o.reshape(_NW, -1, _CHUNK)
    flat_b = flat_b.reshape(_NW, -1, _CHUNK)
    return flat_o, flat_b


_FLAT_O, _FLAT_B = _const_flat_indices()


def _make_gather_kernel(o_chunks, b_chunks):
    mesh = plsc.VectorSubcoreMesh(core_axis_name="c", subcore_axis_name="s")

    @functools.partial(
        pl.kernel,
        mesh=mesh,
        out_type=[
            jax.ShapeDtypeStruct((_NW, o_chunks, _CHUNK), jnp.float32),
            jax.ShapeDtypeStruct((_NW, b_chunks, _CHUNK), jnp.float32),
        ],
        scratch_types=[
            pltpu.VMEM((o_chunks, _CHUNK), jnp.int32),
            pltpu.VMEM((o_chunks, _CHUNK), jnp.float32),
            pltpu.VMEM((b_chunks, _CHUNK), jnp.int32),
            pltpu.VMEM((b_chunks, _CHUNK), jnp.float32),
            pltpu.SemaphoreType.DMA,
        ],
    )
    def gather_k(omega_hbm, oidx_hbm, border_hbm, bidx_hbm,
                 out_o, out_b, oidx_v, orows_v, bidx_v, brows_v, sem):
        wid = lax.axis_index("s") * _NC + lax.axis_index("c")
        pltpu.sync_copy(oidx_hbm.at[wid], oidx_v)
        pltpu.sync_copy(bidx_hbm.at[wid], bidx_v)
        handles = []
        for j in range(o_chunks):
            handles.append(
                pltpu.async_copy(omega_hbm.at[oidx_v.at[j]], orows_v.at[j], sem))
        for j in range(b_chunks):
            handles.append(
                pltpu.async_copy(border_hbm.at[bidx_v.at[j]], brows_v.at[j], sem))
        for h in handles:
            h.wait()
        pltpu.sync_copy(orows_v, out_o.at[wid])
        pltpu.sync_copy(brows_v, out_b.at[wid])

    return gather_k


def kernel(omega, omega_border):
    flat_o, flat_b = _FLAT_O, _FLAT_B
    o_chunks, b_chunks = flat_o.shape[1], flat_b.shape[1]
    gather_k = _make_gather_kernel(o_chunks, b_chunks)
    omega_flat = omega.reshape(_N // 128, 128, _DIM).transpose(0, 2, 1).reshape(-1)
    border_flat = omega_border.transpose(1, 2, 0).reshape(-1)
    out_o, out_b = gather_k(
        omega_flat,
        jnp.asarray(flat_o),
        border_flat,
        jnp.asarray(flat_b),
    )
    omega_batch = out_o.reshape(_OMEGA_BATCH, _DIM)
    border_batch = out_b.reshape(_BORDER_BATCH, _DIM, _N_FACETS)
    return (omega_batch, border_batch)


# speedup vs baseline: 33.3014x; 1.2006x over previous
import functools

import numpy as np
import jax
import jax.numpy as jnp
from jax import lax
from jax.experimental import pallas as pl
from jax.experimental.pallas import tpu as pltpu
from jax.experimental.pallas import tpu_sc as plsc

_N = 4000000
_DIM = 2
_OMEGA_BATCH = 32768
_BORDER_BATCH = 1024
_N_FACETS = 2 * _DIM
_FACET_PTS = 40000 // _N_FACETS

_NC, _NS = 2, 16
_NW = _NC * _NS
_CHUNK = 128


_ROT_A = (13, 15, 26, 6)
_ROT_B = (17, 29, 16, 24)


def _rotl(x, d):
    return (x << np.uint32(d)) | (x >> np.uint32(32 - d))


def _tf2x32(k1, k2, x0, x1):
    k1 = np.uint32(k1)
    k2 = np.uint32(k2)
    ks = (k1, k2, np.uint32(k1 ^ k2 ^ np.uint32(0x1BD11BDA)))
    x0 = (x0 + ks[0]).astype(np.uint32)
    x1 = (x1 + ks[1]).astype(np.uint32)
    rots = (_ROT_A, _ROT_B, _ROT_A, _ROT_B, _ROT_A)
    for i in range(5):
        for r in rots[i]:
            x0 = (x0 + x1).astype(np.uint32)
            x1 = _rotl(x1, r)
            x1 = x1 ^ x0
        x0 = (x0 + ks[(i + 1) % 3]).astype(np.uint32)
        x1 = (x1 + ks[(i + 2) % 3] + np.uint32(i + 1)).astype(np.uint32)
    return x0, x1


def _np_split(key, n):
    lo = np.arange(n, dtype=np.uint32)
    hi = np.zeros(n, dtype=np.uint32)
    b1, b2 = _tf2x32(key[0], key[1], hi, lo)
    return np.stack([b1, b2], axis=1)


def _np_bits32(key, n):
    lo = np.arange(n, dtype=np.uint32)
    hi = np.zeros(n, dtype=np.uint32)
    b1, b2 = _tf2x32(key[0], key[1], hi, lo)
    return b1 ^ b2


def _perm_prefix_np(subkey, n, batch, num_rounds):
    k = subkey
    argsorts = []
    for _ in range(num_rounds):
        pair = _np_split(k, 2)
        k, sk = pair[0], pair[1]
        bits = _np_bits32(sk, n)
        argsorts.append(np.argsort(bits, kind="stable"))
    idx = argsorts[-1][:batch]
    for s in reversed(argsorts[:-1]):
        idx = s[idx]
    return idx


def _const_flat_indices():
    key = np.array([0, 42], dtype=np.uint32)
    key, k_in, k_b = _np_split(key, 3)
    sub = _np_split(k_in, 2)[1]
    idx_o = _perm_prefix_np(sub, _N, _OMEGA_BATCH, 3)
    subb = _np_split(k_b, 2)[1]
    idx_b = _perm_prefix_np(subb, _FACET_PTS, _BORDER_BATCH, 2)
    i64 = idx_o.astype(np.int64)
    src_o = ((i64 // 128)[:, None] * 256
             + np.arange(_DIM, dtype=np.int64) * 128
             + (i64 % 128)[:, None])
    flat_o = (src_o.T.reshape(_DIM, -1, _CHUNK)
              .transpose(1, 0, 2)
              .reshape(-1).astype(np.int32))
    b64 = idx_b.astype(np.int64)
    src_b = (np.arange(_DIM, dtype=np.int64)[:, None, None] * (4 * _FACET_PTS)
             + np.arange(_N_FACETS, dtype=np.int64)[None, :, None] * _FACET_PTS
             + b64[None, None, :])
    flat_b = (src_b.reshape(_DIM, _N_FACETS, -1, _CHUNK)
              .transpose(0, 2, 1, 3)
              .reshape(-1).astype(np.int32))
    flat_o = flat_o.reshape(_NW, -1, _CHUNK)
    flat_b = flat_b.reshape(_NW, -1, _CHUNK)
    return flat_o, flat_b


_FLAT_O, _FLAT_B = _const_flat_indices()


def _make_gather_kernel(o_chunks, b_chunks):
    mesh = plsc.VectorSubcoreMesh(core_axis_name="c", subcore_axis_name="s")

    @functools.partial(
        pl.kernel,
        mesh=mesh,
        out_type=[
            jax.ShapeDtypeStruct((_NW, o_chunks, _CHUNK), jnp.float32),
            jax.ShapeDtypeStruct((_NW, b_chunks, _CHUNK), jnp.float32),
        ],
        scratch_types=[
            pltpu.VMEM((o_chunks, _CHUNK), jnp.int32),
            pltpu.VMEM((o_chunks, _CHUNK), jnp.float32),
            pltpu.VMEM((b_chunks, _CHUNK), jnp.int32),
            pltpu.VMEM((b_chunks, _CHUNK), jnp.float32),
            pltpu.SemaphoreType.DMA,
        ],
    )
    def gather_k(omega_hbm, oidx_hbm, border_hbm, bidx_hbm,
                 out_o, out_b, oidx_v, orows_v, bidx_v, brows_v, sem):
        wid = lax.axis_index("s") * _NC + lax.axis_index("c")
        pltpu.sync_copy(oidx_hbm.at[wid], oidx_v)
        pltpu.sync_copy(bidx_hbm.at[wid], bidx_v)
        handles = []
        for j in range(o_chunks):
            handles.append(
                pltpu.async_copy(omega_hbm.at[oidx_v.at[j]], orows_v.at[j], sem))
        for j in range(b_chunks):
            handles.append(
                pltpu.async_copy(border_hbm.at[bidx_v.at[j]], brows_v.at[j], sem))
        for h in handles:
            h.wait()
        pltpu.sync_copy(orows_v, out_o.at[wid])
        pltpu.sync_copy(brows_v, out_b.at[wid])

    return gather_k


def kernel(omega, omega_border):
    flat_o, flat_b = _FLAT_O, _FLAT_B
    o_chunks, b_chunks = flat_o.shape[1], flat_b.shape[1]
    gather_k = _make_gather_kernel(o_chunks, b_chunks)
    omega_flat = omega.reshape(_N // 128, 128, _DIM).transpose(0, 2, 1).reshape(-1)
    border_flat = omega_border.transpose(1, 2, 0).reshape(-1)
    out_o, out_b = gather_k(
        omega_flat,
        jnp.asarray(flat_o),
        border_flat,
        jnp.asarray(flat_b),
    )
    omega_batch = (out_o.reshape(_OMEGA_BATCH // _CHUNK, _DIM, _CHUNK)
                   .transpose(0, 2, 1).reshape(_OMEGA_BATCH, _DIM))
    border_batch = (out_b.reshape(_DIM, _BORDER_BATCH // _CHUNK, _N_FACETS, _CHUNK)
                    .transpose(1, 3, 0, 2).reshape(_BORDER_BATCH, _DIM, _N_FACETS))
    return (omega_batch, border_batch)
